# baseline (speedup 1.0000x reference)
import functools

import jax
import jax.numpy as jnp
from jax import lax
from jax.experimental import pallas as pl
from jax.experimental.pallas import tpu as pltpu
from jax.experimental.pallas import tpu_sc as plsc

N = 10000
E = 320000
D = 128

NC, NS, LANES = 2, 16, 16
NW = NC * NS
CHUNK = 128
N_PAD = 10112
ROWS_PER_TILE = N_PAD // NS
CNT_W = 16

NBUF = 4
_N_CHUNKS = -(-E // CHUNK)
_CPT = -(-_N_CHUNKS // (NS * NBUF)) * NBUF
E_PAD = _CPT * NS * CHUNK


DH = D // NC


def _make_agg(with_cnt: bool):
    mesh = plsc.VectorSubcoreMesh(core_axis_name="c", subcore_axis_name="s")
    agg_ty = jax.ShapeDtypeStruct((NC, N_PAD, DH), jnp.float32)
    if with_cnt:
        out_type = [agg_ty, jax.ShapeDtypeStruct((NS, N_PAD), jnp.float32)]
    else:
        out_type = agg_ty
    scratch_types = [
        pltpu.VMEM((_CPT, CHUNK), jnp.int32),
        pltpu.VMEM((_CPT, CHUNK), jnp.int32),
        [pltpu.VMEM((CHUNK,), jnp.int32) for _ in range(NBUF)],
        [pltpu.VMEM((CHUNK,), jnp.int32) for _ in range(NBUF)],
        [pltpu.VMEM((CHUNK, DH), jnp.float32) for _ in range(NBUF)],
        pltpu.VMEM((N_PAD,), jnp.float32),
        pltpu.VMEM_SHARED((N_PAD, DH), jnp.float32),
        [pltpu.SemaphoreType.DMA for _ in range(NBUF)],
        [pltpu.SemaphoreType.DMA for _ in range(NBUF)],
    ]

    def body(x0_hbm, x1_hbm, src_hbm, dst_hbm, *rest):
        if with_cnt:
            out_hbm, cnt_hbm = rest[0], rest[1]
            rest = rest[2:]
        else:
            out_hbm = rest[0]
            cnt_hbm = None
            rest = rest[1:]
        (srcs_v, dsts_v, src_v, dst_v, rows, cnt_t, acc_sh,
         gsem, ssem) = rest

        c = lax.axis_index("c")
        s = lax.axis_index("s")
        chunk0 = s * _CPT

        pltpu.sync_copy(src_hbm.at[pl.ds(chunk0, _CPT)], srcs_v)
        pltpu.sync_copy(dst_hbm.at[pl.ds(chunk0, _CPT)], dsts_v)

        def _fill_row(i, _):
            for k in range(DH // LANES):
                rows[0][i, pl.ds(k * LANES, LANES)] = jnp.zeros((LANES,), jnp.float32)
            return 0
        lax.fori_loop(0, CHUNK, _fill_row, 0)
        if with_cnt:
            def _fill_cnt(i, _):
                cnt_t[pl.ds(i * LANES, LANES)] = jnp.zeros((LANES,), jnp.float32)
                return 0
            lax.fori_loop(0, N_PAD // LANES, _fill_cnt, 0)

        base_r = s * ROWS_PER_TILE
        n_full = ROWS_PER_TILE // CHUNK
        rem = ROWS_PER_TILE - n_full * CHUNK
        def _zero_acc(i, _):
            pltpu.sync_copy(rows[0], acc_sh.at[pl.ds(base_r + i * CHUNK, CHUNK)])
            return 0
        lax.fori_loop(0, n_full, _zero_acc, 0)
        if rem:
            pltpu.sync_copy(rows[0].at[pl.ds(0, rem)],
                            acc_sh.at[pl.ds(base_r + n_full * CHUNK, rem)])

        plsc.subcore_barrier()

        def _gissue(j, b):
            for k in range(CHUNK // LANES):
                src_v[b][pl.ds(k * LANES, LANES)] = srcs_v[j, pl.ds(k * LANES, LANES)]
            @pl.when(c == 0)
            def _():
                pltpu.async_copy(x0_hbm.at[src_v[b]], rows[b], gsem[b])
            @pl.when(c == 1)
            def _():
                pltpu.async_copy(x1_hbm.at[src_v[b]], rows[b], gsem[b])

        for b in range(NBUF):
            _gissue(b, b)

        ones16 = jnp.ones((LANES,), jnp.float32)

        def _round(i, _):
            for b in range(NBUF):
                j = i * NBUF + b
                for k in range(CHUNK // LANES):
                    dv = dsts_v[j, pl.ds(k * LANES, LANES)]
                    dst_v[b][pl.ds(k * LANES, LANES)] = dv
                    if with_cnt:
                        @pl.when(c == 0)
                        def _():
                            plsc.addupdate_scatter(cnt_t, [dv], ones16)
                pltpu.make_async_copy(x0_hbm.at[src_v[b]], rows[b],
                                      gsem[b]).wait()
                pltpu.sync_copy(rows[b], acc_sh.at[dst_v[b]], add=True)
                @pl.when(j + NBUF < _CPT)
                def _():
                    _gissue(j + NBUF, b)
            return 0
        lax.fori_loop(0, _CPT // NBUF, _round, 0)

        plsc.subcore_barrier()

        pltpu.sync_copy(acc_sh.at[pl.ds(base_r, ROWS_PER_TILE)],
                        out_hbm.at[c, pl.ds(base_r, ROWS_PER_TILE)])
        if with_cnt:
            @pl.when(c == 0)
            def _():
                pltpu.sync_copy(cnt_t, cnt_hbm.at[s])

    return pl.kernel(body, out_type=out_type, mesh=mesh,
                     scratch_types=scratch_types,
                     compiler_params=pltpu.CompilerParams(
                         use_tc_tiling_on_sc=False,
                         needs_layout_passes=False))


_agg_with_cnt = _make_agg(True)
_agg_no_cnt = _make_agg(False)

_ROWS_BLK = 1264


def _dense_body(parts_ref, cnt_ref, x_ref, wn_ref, ws_ref, b_ref, o_ref, *, act):
    agg = jnp.concatenate([parts_ref[0], parts_ref[1]], axis=1)
    cnt = jnp.sum(cnt_ref[...], axis=0)
    mean = agg / jnp.maximum(cnt, 1.0)
    y = (jnp.dot(mean, wn_ref[...], preferred_element_type=jnp.float32)
         + jnp.dot(x_ref[...], ws_ref[...], preferred_element_type=jnp.float32)
         + b_ref[...])
    if act == "elu":
        o_ref[...] = jnp.where(y > 0, y, jnp.exp(jnp.minimum(y, 0.0)) - 1.0)
    else:
        m = jnp.max(y, axis=1, keepdims=True)
        lse = jnp.log(jnp.sum(jnp.exp(y - m), axis=1, keepdims=True)) + m
        o_ref[...] = y - lse


def _dense(parts, cnt, x, w_neigh, w_self, b, act):
    grid = N_PAD // _ROWS_BLK
    return pl.pallas_call(
        functools.partial(_dense_body, act=act),
        grid=(grid,),
        in_specs=[
            pl.BlockSpec((NC, _ROWS_BLK, DH), lambda i: (0, i, 0)),
            pl.BlockSpec((NS, _ROWS_BLK, 1), lambda i: (0, i, 0)),
            pl.BlockSpec((_ROWS_BLK, D), lambda i: (i, 0)),
            pl.BlockSpec((D, D), lambda i: (0, 0)),
            pl.BlockSpec((D, D), lambda i: (0, 0)),
            pl.BlockSpec((1, D), lambda i: (0, 0)),
        ],
        out_specs=pl.BlockSpec((_ROWS_BLK, D), lambda i: (i, 0)),
        out_shape=jax.ShapeDtypeStruct((N_PAD, D), jnp.float32),
    )(parts, cnt, x, w_neigh, w_self, b)


def kernel(x, edge_index, W1_neigh, W1_self, b1, W2_neigh, W2_self, b2):
    src = edge_index[0]
    dst = edge_index[1]
    pad = E_PAD - E
    src_p = jnp.concatenate([src, jnp.zeros((pad,), jnp.int32)]).reshape(-1, CHUNK)
    dst_p = jnp.concatenate([dst, jnp.full((pad,), N, jnp.int32)]).reshape(-1, CHUNK)
    x_p = jnp.pad(x, ((0, N_PAD - N), (0, 0)))

    parts1, cnt = _agg_with_cnt(x_p[:, :DH], x_p[:, DH:], src_p, dst_p)
    cnt3 = cnt[:, :, None]
    h = _dense(parts1, cnt3, x_p, W1_neigh, W1_self, b1.reshape(1, D), "elu")
    parts2 = _agg_no_cnt(h[:, :DH], h[:, DH:], src_p, dst_p)
    out = _dense(parts2, cnt3, h, W2_neigh, W2_self, b2.reshape(1, D), "lsm")
    return out[:N]


# speedup vs baseline: 1.3339x; 1.3339x over previous
import functools

import jax
import jax.numpy as jnp
from jax import lax
from jax.experimental import pallas as pl
from jax.experimental.pallas import tpu as pltpu
from jax.experimental.pallas import tpu_sc as plsc

N = 10000
E = 320000
D = 128

NC, NS, LANES = 2, 16, 16
NW = NC * NS
CHUNK = 128
N_PAD = 10112
ROWS_PER_TILE = N_PAD // NS
CNT_W = 16

NBUF = 4
_N_CHUNKS = -(-E // CHUNK)
_CPT = -(-_N_CHUNKS // (NS * NBUF)) * NBUF
E_PAD = _CPT * NS * CHUNK


DH = D // NC


def _make_agg(with_cnt: bool):
    mesh = plsc.VectorSubcoreMesh(core_axis_name="c", subcore_axis_name="s")
    agg_ty = jax.ShapeDtypeStruct((NC, N_PAD, DH), jnp.float32)
    if with_cnt:
        out_type = [agg_ty, jax.ShapeDtypeStruct((N_PAD, CNT_W), jnp.float32)]
    else:
        out_type = agg_ty
    scratch_types = [
        pltpu.VMEM((_CPT, CHUNK), jnp.int32),
        pltpu.VMEM((_CPT, CHUNK), jnp.int32),
        [pltpu.VMEM((CHUNK,), jnp.int32) for _ in range(NBUF)],
        [pltpu.VMEM((CHUNK,), jnp.int32) for _ in range(NBUF)],
        [pltpu.VMEM((CHUNK, DH), jnp.float32) for _ in range(NBUF)],
        pltpu.VMEM((CHUNK, CNT_W), jnp.float32),
        pltpu.VMEM((CHUNK, CNT_W), jnp.float32),
        pltpu.VMEM_SHARED((N_PAD, DH), jnp.float32),
        pltpu.VMEM_SHARED((N_PAD, CNT_W), jnp.float32),
        [pltpu.SemaphoreType.DMA for _ in range(NBUF)],
        [pltpu.SemaphoreType.DMA for _ in range(NBUF)],
        pltpu.SemaphoreType.DMA,
    ]

    def body(x0_hbm, x1_hbm, src_hbm, dst_hbm, *rest):
        if with_cnt:
            out_hbm, cnt_hbm = rest[0], rest[1]
            rest = rest[2:]
        else:
            out_hbm = rest[0]
            cnt_hbm = None
            rest = rest[1:]
        (srcs_v, dsts_v, src_v, dst_v, rows, ones_v, zc_v, acc_sh, cnt_sh,
         gsem, ssem, csem) = rest

        c = lax.axis_index("c")
        s = lax.axis_index("s")
        chunk0 = s * _CPT

        pltpu.sync_copy(src_hbm.at[pl.ds(chunk0, _CPT)], srcs_v)
        pltpu.sync_copy(dst_hbm.at[pl.ds(chunk0, _CPT)], dsts_v)

        def _fill_row(i, _):
            for k in range(DH // LANES):
                rows[0][i, pl.ds(k * LANES, LANES)] = jnp.zeros((LANES,), jnp.float32)
            for k in range(CNT_W // LANES):
                ones_v[i, pl.ds(k * LANES, LANES)] = jnp.ones((LANES,), jnp.float32)
                zc_v[i, pl.ds(k * LANES, LANES)] = jnp.zeros((LANES,), jnp.float32)
            return 0
        lax.fori_loop(0, CHUNK, _fill_row, 0)

        base_r = s * ROWS_PER_TILE
        n_full = ROWS_PER_TILE // CHUNK
        rem = ROWS_PER_TILE - n_full * CHUNK
        def _zero_acc(i, _):
            pltpu.sync_copy(rows[0], acc_sh.at[pl.ds(base_r + i * CHUNK, CHUNK)])
            pltpu.sync_copy(zc_v, cnt_sh.at[pl.ds(base_r + i * CHUNK, CHUNK)])
            return 0
        lax.fori_loop(0, n_full, _zero_acc, 0)
        if rem:
            pltpu.sync_copy(rows[0].at[pl.ds(0, rem)],
                            acc_sh.at[pl.ds(base_r + n_full * CHUNK, rem)])
            pltpu.sync_copy(zc_v.at[pl.ds(0, rem)],
                            cnt_sh.at[pl.ds(base_r + n_full * CHUNK, rem)])

        plsc.subcore_barrier()

        def _gissue(j, b):
            for k in range(CHUNK // LANES):
                src_v[b][pl.ds(k * LANES, LANES)] = srcs_v[j, pl.ds(k * LANES, LANES)]
            @pl.when(c == 0)
            def _():
                pltpu.async_copy(x0_hbm.at[src_v[b]], rows[b], gsem[b])
            @pl.when(c == 1)
            def _():
                pltpu.async_copy(x1_hbm.at[src_v[b]], rows[b], gsem[b])

        for b in range(NBUF):
            _gissue(b, b)

        def _round(i, _):
            for b in range(NBUF):
                j = i * NBUF + b
                for k in range(CHUNK // LANES):
                    dst_v[b][pl.ds(k * LANES, LANES)] = dsts_v[j, pl.ds(k * LANES, LANES)]
                pltpu.make_async_copy(x0_hbm.at[src_v[b]], rows[b],
                                      gsem[b]).wait()
                pltpu.sync_copy(rows[b], acc_sh.at[dst_v[b]], add=True)
                if with_cnt:
                    @pl.when(c == 0)
                    def _():
                        pltpu.sync_copy(ones_v, cnt_sh.at[dst_v[b]], add=True)
                @pl.when(j + NBUF < _CPT)
                def _():
                    _gissue(j + NBUF, b)
            return 0
        lax.fori_loop(0, _CPT // NBUF, _round, 0)

        plsc.subcore_barrier()

        pltpu.sync_copy(acc_sh.at[pl.ds(base_r, ROWS_PER_TILE)],
                        out_hbm.at[c, pl.ds(base_r, ROWS_PER_TILE)])
        if with_cnt:
            @pl.when(c == 0)
            def _():
                pltpu.sync_copy(cnt_sh.at[pl.ds(base_r, ROWS_PER_TILE)],
                                cnt_hbm.at[pl.ds(base_r, ROWS_PER_TILE)])

    return pl.kernel(body, out_type=out_type, mesh=mesh,
                     scratch_types=scratch_types,
                     compiler_params=pltpu.CompilerParams(
                         use_tc_tiling_on_sc=False))


_agg_with_cnt = _make_agg(True)
_agg_no_cnt = _make_agg(False)

_ROWS_BLK = 1264


def _dense_body(parts_ref, cnt_ref, xa_ref, xb_ref, wn_ref, ws_ref, b_ref,
                *out_refs, act):
    agg = jnp.concatenate([parts_ref[0], parts_ref[1]], axis=1)
    cnt = cnt_ref[:, 0:1]
    mean = agg / jnp.maximum(cnt, 1.0)
    xcat = jnp.concatenate([xa_ref[...], xb_ref[...]], axis=1)
    y = (jnp.dot(mean, wn_ref[...], preferred_element_type=jnp.float32)
         + jnp.dot(xcat, ws_ref[...], preferred_element_type=jnp.float32)
         + b_ref[...])
    if act == "elu":
        h = jnp.where(y > 0, y, jnp.exp(jnp.minimum(y, 0.0)) - 1.0)
        out_refs[0][...] = h[:, :DH]
        out_refs[1][...] = h[:, DH:]
    else:
        m = jnp.max(y, axis=1, keepdims=True)
        lse = jnp.log(jnp.sum(jnp.exp(y - m), axis=1, keepdims=True)) + m
        out_refs[0][...] = y - lse


def _dense(parts, cnt, xa, xb, w_neigh, w_self, b, act):
    grid = N_PAD // _ROWS_BLK
    if act == "elu":
        out_specs = [pl.BlockSpec((_ROWS_BLK, DH), lambda i: (i, 0)),
                     pl.BlockSpec((_ROWS_BLK, DH), lambda i: (i, 0))]
        out_shape = [jax.ShapeDtypeStruct((N_PAD, DH), jnp.float32),
                     jax.ShapeDtypeStruct((N_PAD, DH), jnp.float32)]
    else:
        out_specs = pl.BlockSpec((_ROWS_BLK, D), lambda i: (i, 0))
        out_shape = jax.ShapeDtypeStruct((N_PAD, D), jnp.float32)
    return pl.pallas_call(
        functools.partial(_dense_body, act=act),
        grid=(grid,),
        in_specs=[
            pl.BlockSpec((NC, _ROWS_BLK, DH), lambda i: (0, i, 0)),
            pl.BlockSpec((_ROWS_BLK, CNT_W), lambda i: (i, 0)),
            pl.BlockSpec((_ROWS_BLK, DH), lambda i: (i, 0)),
            pl.BlockSpec((_ROWS_BLK, DH), lambda i: (i, 0)),
            pl.BlockSpec((D, D), lambda i: (0, 0)),
            pl.BlockSpec((D, D), lambda i: (0, 0)),
            pl.BlockSpec((1, D), lambda i: (0, 0)),
        ],
        out_specs=out_specs,
        out_shape=out_shape,
    )(parts, cnt, xa, xb, w_neigh, w_self, b)


def kernel(x, edge_index, W1_neigh, W1_self, b1, W2_neigh, W2_self, b2):
    src = edge_index[0]
    dst = edge_index[1]
    pad = E_PAD - E
    src_p = jnp.concatenate([src, jnp.zeros((pad,), jnp.int32)]).reshape(-1, CHUNK)
    dst_p = jnp.concatenate([dst, jnp.full((pad,), N, jnp.int32)]).reshape(-1, CHUNK)
    x0 = jnp.pad(x[:, :DH], ((0, N_PAD - N), (0, 0)))
    x1 = jnp.pad(x[:, DH:], ((0, N_PAD - N), (0, 0)))

    parts1, cnt = _agg_with_cnt(x0, x1, src_p, dst_p)
    h0, h1 = _dense(parts1, cnt, x0, x1, W1_neigh, W1_self,
                    b1.reshape(1, D), "elu")
    parts2 = _agg_no_cnt(h0, h1, src_p, dst_p)
    out = _dense(parts2, cnt, h0, h1, W2_neigh, W2_self,
                 b2.reshape(1, D), "lsm")
    return out[:N]
